# Initial kernel scaffold; baseline (speedup 1.0000x reference)
import jax
import jax.numpy as jnp
from jax.experimental import pallas as pl


def kernel(idx, table, W, b):
    raise NotImplementedError("write your pallas kernel here")


# speedup vs baseline: 1.1578x; 1.1578x over previous
import functools

import jax
import jax.numpy as jnp
from jax import lax
from jax.experimental import pallas as pl
from jax.experimental.pallas import tpu as pltpu
from jax.experimental.pallas import tpu_sc as plsc

PHI = 1.61803398875

NC, NS = 2, 16
NW = NC * NS
CHUNK = 128
K = 4
GROUP = CHUNK * K


def _gather_sc(table, idx3, total_rows, d):
    nchunks = idx3.shape[1]
    ngroups = nchunks // K
    rows_per_w = nchunks * CHUNK

    mesh = plsc.VectorSubcoreMesh(core_axis_name="c", subcore_axis_name="s")

    @functools.partial(
        pl.kernel,
        out_type=jax.ShapeDtypeStruct((total_rows, d), jnp.float32),
        mesh=mesh,
        scratch_types=[
            pltpu.VMEM((nchunks, CHUNK), jnp.int32),
            pltpu.VMEM((GROUP, d), jnp.float32),
            pltpu.SemaphoreType.DMA,
        ],
        compiler_params=pltpu.CompilerParams(use_tc_tiling_on_sc=False),
    )
    def gather_kernel(table_hbm, idx_hbm, out_hbm, idx_v, buf, sem):
        wid = lax.axis_index("s") * NC + lax.axis_index("c")
        base = wid * rows_per_w
        pltpu.sync_copy(idx_hbm.at[wid], idx_v)

        def body(g, carry):
            copies = [
                pltpu.async_copy(
                    table_hbm.at[idx_v.at[g * K + j]],
                    buf.at[pl.ds(j * CHUNK, CHUNK)],
                    sem,
                )
                for j in range(K)
            ]
            for c in copies:
                c.wait()
            pltpu.sync_copy(buf, out_hbm.at[pl.ds(base + g * GROUP, GROUP)])
            return carry

        lax.fori_loop(0, ngroups, body, 0)

    return gather_kernel(table, idx3)


def _linear_tc(x, wt, b2, m, d):
    blk = 4096
    grid = m // blk

    def body(x_ref, w_ref, b_ref, o_ref):
        acc = jnp.dot(x_ref[...], w_ref[...], preferred_element_type=jnp.float32)
        o_ref[...] = (acc + b_ref[...]) * PHI

    return pl.pallas_call(
        body,
        grid=(grid,),
        in_specs=[
            pl.BlockSpec((blk, d), lambda i: (i, 0)),
            pl.BlockSpec((d, d), lambda i: (0, 0)),
            pl.BlockSpec((1, d), lambda i: (0, 0)),
        ],
        out_specs=pl.BlockSpec((blk, d), lambda i: (i, 0)),
        out_shape=jax.ShapeDtypeStruct((m, d), jnp.float32),
    )(x, wt, b2)


def kernel(idx, table, W, b):
    bsz, seq = idx.shape
    v, d = table.shape
    m = bsz * seq
    idx3 = idx.reshape(NW, m // (NW * CHUNK), CHUNK).astype(jnp.int32)
    base2d = _gather_sc(table, idx3, m, d)
    out = _linear_tc(base2d, W.T, b.reshape(1, d), m, d)
    return out.reshape(bsz, seq, d)


# speedup vs baseline: 1.4483x; 1.2509x over previous
import functools

import jax
import jax.numpy as jnp
from jax import lax
from jax.experimental import pallas as pl
from jax.experimental.pallas import tpu as pltpu
from jax.experimental.pallas import tpu_sc as plsc

PHI = 1.61803398875

NC, NS = 2, 16
NW = NC * NS
CHUNK = 128
K = 4
GROUP = CHUNK * K


def _transform_tc(table, w2, b2, v, d2):
    blk = 8000
    grid = v // blk
    d = table.shape[1]

    def body(x_ref, w_ref, b_ref, o_ref):
        acc = jnp.dot(x_ref[...], w_ref[...], preferred_element_type=jnp.float32)
        o_ref[...] = (acc + b_ref[...]) * PHI

    return pl.pallas_call(
        body,
        grid=(grid,),
        in_specs=[
            pl.BlockSpec((blk, d), lambda i: (i, 0)),
            pl.BlockSpec((d, d2), lambda i: (0, 0)),
            pl.BlockSpec((1, d2), lambda i: (0, 0)),
        ],
        out_specs=pl.BlockSpec((blk, d2), lambda i: (i, 0)),
        out_shape=jax.ShapeDtypeStruct((v, d2), jnp.float32),
    )(table, w2, b2)


def _gather_sc(t2, idx3, total_rows, d2):
    nchunks = idx3.shape[1]
    ngroups = nchunks // K
    rows_per_w = nchunks * CHUNK

    mesh = plsc.VectorSubcoreMesh(core_axis_name="c", subcore_axis_name="s")

    @functools.partial(
        pl.kernel,
        out_type=jax.ShapeDtypeStruct((total_rows, d2), jnp.float32),
        mesh=mesh,
        scratch_types=[
            pltpu.VMEM((nchunks, CHUNK), jnp.int32),
            pltpu.VMEM((GROUP, d2), jnp.float32),
            pltpu.SemaphoreType.DMA,
        ],
        compiler_params=pltpu.CompilerParams(use_tc_tiling_on_sc=True),
    )
    def gather_kernel(t2_hbm, idx_hbm, out_hbm, idx_v, buf, sem):
        wid = lax.axis_index("s") * NC + lax.axis_index("c")
        base = wid * rows_per_w
        pltpu.sync_copy(idx_hbm.at[wid], idx_v)

        def body(g, carry):
            copies = [
                pltpu.async_copy(
                    t2_hbm.at[idx_v.at[g * K + j]],
                    buf.at[pl.ds(j * CHUNK, CHUNK)],
                    sem,
                )
                for j in range(K)
            ]
            for c in copies:
                c.wait()
            pltpu.sync_copy(buf, out_hbm.at[pl.ds(base + g * GROUP, GROUP)])
            return carry

        lax.fori_loop(0, ngroups, body, 0)

    return gather_kernel(t2, idx3)


def kernel(idx, table, W, b):
    bsz, seq = idx.shape
    v, d = table.shape
    d2 = 2 * d
    m = bsz * seq
    wt = W.T
    w2 = jnp.concatenate([wt, wt], axis=1)
    b2 = jnp.concatenate([b, b]).reshape(1, d2)
    t2 = _transform_tc(table, w2, b2, v, d2)
    idx3 = idx.reshape(NW, m // (NW * CHUNK), CHUNK).astype(jnp.int32)
    out2 = _gather_sc(t2, idx3, m, d2)
    return out2[:, :d].reshape(bsz, seq, d)


# speedup vs baseline: 2.4309x; 1.6785x over previous
import functools

import jax
import jax.numpy as jnp
from jax import lax
from jax.experimental import pallas as pl
from jax.experimental.pallas import tpu as pltpu
from jax.experimental.pallas import tpu_sc as plsc

PHI = 1.61803398875

NC, NS = 2, 16
NW = NC * NS
CHUNK = 128
K = 4
GROUP = CHUNK * K


def _transform_tc(table_t, w2, b2, v, d, d2):
    blk = 4096
    grid = pl.cdiv(v, blk)

    def body(x_ref, w_ref, b_ref, o_ref):
        acc = lax.dot_general(
            x_ref[...], w_ref[...],
            dimension_numbers=(((0,), (0,)), ((), ())),
            preferred_element_type=jnp.float32,
        )
        o_ref[...] = (acc + b_ref[...]) * PHI

    return pl.pallas_call(
        body,
        grid=(grid,),
        in_specs=[
            pl.BlockSpec((d, blk), lambda i: (0, i)),
            pl.BlockSpec((d, d2), lambda i: (0, 0)),
            pl.BlockSpec((1, d2), lambda i: (0, 0)),
        ],
        out_specs=pl.BlockSpec((blk, d2), lambda i: (i, 0)),
        out_shape=jax.ShapeDtypeStruct((v, d2), jnp.float32),
    )(table_t, w2, b2)


def _gather_sc(t2, idx3, total_rows, d2):
    nchunks = idx3.shape[1]
    ngroups = nchunks // K
    rows_per_w = nchunks * CHUNK

    mesh = plsc.VectorSubcoreMesh(core_axis_name="c", subcore_axis_name="s")

    @functools.partial(
        pl.kernel,
        out_type=jax.ShapeDtypeStruct((total_rows, d2), jnp.float32),
        mesh=mesh,
        scratch_types=[
            pltpu.VMEM((nchunks, CHUNK), jnp.int32),
            pltpu.VMEM((GROUP, d2), jnp.float32),
            pltpu.SemaphoreType.DMA,
        ],
        compiler_params=pltpu.CompilerParams(use_tc_tiling_on_sc=True),
    )
    def gather_kernel(t2_hbm, idx_hbm, out_hbm, idx_v, buf, sem):
        wid = lax.axis_index("s") * NC + lax.axis_index("c")
        base = wid * rows_per_w
        pltpu.sync_copy(idx_hbm.at[wid], idx_v)

        def body(g, carry):
            copies = [
                pltpu.async_copy(
                    t2_hbm.at[idx_v.at[g * K + j]],
                    buf.at[pl.ds(j * CHUNK, CHUNK)],
                    sem,
                )
                for j in range(K)
            ]
            for c in copies:
                c.wait()
            pltpu.sync_copy(buf, out_hbm.at[pl.ds(base + g * GROUP, GROUP)])
            return carry

        lax.fori_loop(0, ngroups, body, 0)

    return gather_kernel(t2, idx3)


def _transpose_tc(g2, ident, seq, d, bsz):
    blk = 4096
    jgrid = bsz // blk

    def body(x_ref, i_ref, o_ref):
        x64 = x_ref[...][:, :d]
        z = lax.dot_general(
            i_ref[...], x64,
            dimension_numbers=(((1,), (1,)), ((), ())),
            preferred_element_type=jnp.float32,
        )
        o_ref[0] = z

    return pl.pallas_call(
        body,
        grid=(seq, jgrid),
        in_specs=[
            pl.BlockSpec((blk, 2 * d), lambda l, j: (l * jgrid + j, 0)),
            pl.BlockSpec((d, d), lambda l, j: (0, 0)),
        ],
        out_specs=pl.BlockSpec((1, d, blk), lambda l, j: (l, 0, j)),
        out_shape=jax.ShapeDtypeStruct((seq, d, bsz), jnp.float32),
    )(g2, ident)


def kernel(idx, table, W, b):
    bsz, seq = idx.shape
    v, d = table.shape
    d2 = 2 * d
    m = bsz * seq
    wt = W.T
    w2 = jnp.concatenate([wt, wt], axis=1)
    b2 = jnp.concatenate([b, b]).reshape(1, d2)
    t2 = _transform_tc(table.T, w2, b2, v, d, d2)
    idx3 = idx.T.reshape(NW, m // (NW * CHUNK), CHUNK).astype(jnp.int32)
    g2 = _gather_sc(t2, idx3, m, d2)
    t3 = _transpose_tc(g2, jnp.eye(d, dtype=jnp.float32), seq, d, bsz)
    return jnp.transpose(t3, (2, 0, 1))


# speedup vs baseline: 2.5696x; 1.0570x over previous
import functools

import jax
import jax.numpy as jnp
from jax import lax
from jax.experimental import pallas as pl
from jax.experimental.pallas import tpu as pltpu
from jax.experimental.pallas import tpu_sc as plsc

PHI = 1.61803398875

NC, NS = 2, 16
NW = NC * NS
CHUNK = 128
K = 4
GROUP = CHUNK * K


def _transform_tc(table_t, w2, b2, v, d, d2):
    blk = 4096
    grid = pl.cdiv(v, blk)

    def body(x_ref, w_ref, b_ref, o_ref):
        acc = lax.dot_general(
            x_ref[...], w_ref[...],
            dimension_numbers=(((0,), (0,)), ((), ())),
            preferred_element_type=jnp.float32,
        )
        o_ref[...] = (acc + b_ref[...]) * PHI

    return pl.pallas_call(
        body,
        grid=(grid,),
        in_specs=[
            pl.BlockSpec((d, blk), lambda i: (0, i)),
            pl.BlockSpec((d, d2), lambda i: (0, 0)),
            pl.BlockSpec((1, d2), lambda i: (0, 0)),
        ],
        out_specs=pl.BlockSpec((blk, d2), lambda i: (i, 0)),
        out_shape=jax.ShapeDtypeStruct((v, d2), jnp.float32),
    )(table_t, w2, b2)


def _gather_sc(t2, idx3, total_rows, d2):
    nchunks = idx3.shape[1]
    ngroups = nchunks // K
    rows_per_w = nchunks * CHUNK

    mesh = plsc.VectorSubcoreMesh(core_axis_name="c", subcore_axis_name="s")

    @functools.partial(
        pl.kernel,
        out_type=jax.ShapeDtypeStruct((total_rows, d2), jnp.float32),
        mesh=mesh,
        scratch_types=[
            pltpu.VMEM((nchunks, CHUNK), jnp.int32),
            pltpu.VMEM((GROUP, d2), jnp.float32),
            pltpu.SemaphoreType.DMA,
        ],
        compiler_params=pltpu.CompilerParams(use_tc_tiling_on_sc=True),
    )
    def gather_kernel(t2_hbm, idx_hbm, out_hbm, idx_v, buf, sem):
        wid = lax.axis_index("s") * NC + lax.axis_index("c")
        base = wid * rows_per_w
        pltpu.sync_copy(idx_hbm.at[wid], idx_v)

        def body(g, carry):
            copies = [
                pltpu.async_copy(
                    t2_hbm.at[idx_v.at[g * K + j]],
                    buf.at[pl.ds(j * CHUNK, CHUNK)],
                    sem,
                )
                for j in range(K)
            ]
            for c in copies:
                c.wait()
            pltpu.sync_copy(buf, out_hbm.at[pl.ds(base + g * GROUP, GROUP)])
            return carry

        lax.fori_loop(0, ngroups, body, 0)

    return gather_kernel(t2, idx3)


def _transpose_tc(g2_s, ident, acc, l0, nl, seq, d, bsz):
    blk = 4096
    jgrid = bsz // blk

    def body(*refs):
        x_ref, i_ref, o_ref = refs[-3], refs[-2], refs[-1]
        x64 = x_ref[...][:, :d]
        z = lax.dot_general(
            i_ref[...], x64,
            dimension_numbers=(((1,), (1,)), ((), ())),
            preferred_element_type=jnp.float32,
        )
        o_ref[0] = z

    in_specs = [
        pl.BlockSpec((blk, 2 * d), lambda l, j: (l * jgrid + j, 0)),
        pl.BlockSpec((d, d), lambda l, j: (0, 0)),
    ]
    args = (g2_s, ident)
    aliases = {}
    if acc is not None:
        in_specs = [pl.BlockSpec(memory_space=pl.ANY)] + in_specs
        args = (acc,) + args
        aliases = {0: 0}

    return pl.pallas_call(
        body,
        grid=(nl, jgrid),
        in_specs=in_specs,
        out_specs=pl.BlockSpec((1, d, blk), lambda l, j: (l0 + l, 0, j)),
        out_shape=jax.ShapeDtypeStruct((seq, d, bsz), jnp.float32),
        input_output_aliases=aliases,
    )(*args)


def kernel(idx, table, W, b):
    bsz, seq = idx.shape
    v, d = table.shape
    d2 = 2 * d
    m = bsz * seq
    wt = W.T
    w2 = jnp.concatenate([wt, wt], axis=1)
    b2 = jnp.concatenate([b, b]).reshape(1, d2)
    t2 = _transform_tc(table.T, w2, b2, v, d, d2)
    nsl = 5
    nl = seq // nsl
    rows_s = nl * bsz
    idx4 = idx.T.reshape(nsl, NW, rows_s // (NW * CHUNK), CHUNK).astype(jnp.int32)
    ident = jnp.eye(d, dtype=jnp.float32)
    acc = None
    for s in range(nsl):
        g2_s = _gather_sc(t2, idx4[s], rows_s, d2)
        acc = _transpose_tc(g2_s, ident, acc, s * nl, nl, seq, d, bsz)
    return jnp.transpose(acc, (2, 0, 1))


# speedup vs baseline: 2.6023x; 1.0127x over previous
import functools

import jax
import jax.numpy as jnp
from jax import lax
from jax.experimental import pallas as pl
from jax.experimental.pallas import tpu as pltpu
from jax.experimental.pallas import tpu_sc as plsc

PHI = 1.61803398875

NC, NS = 2, 16
NW = NC * NS
CHUNK = 128
K = 4
GROUP = CHUNK * K


def _transform_tc(table_t, w2, b2, v, d, d2):
    blk = 4096
    grid = pl.cdiv(v, blk)

    def body(x_ref, w_ref, b_ref, o_ref):
        acc = lax.dot_general(
            x_ref[...], w_ref[...],
            dimension_numbers=(((0,), (0,)), ((), ())),
            preferred_element_type=jnp.float32,
        )
        o_ref[...] = (acc + b_ref[...]) * PHI

    return pl.pallas_call(
        body,
        grid=(grid,),
        in_specs=[
            pl.BlockSpec((d, blk), lambda i: (0, i)),
            pl.BlockSpec((d, d2), lambda i: (0, 0)),
            pl.BlockSpec((1, d2), lambda i: (0, 0)),
        ],
        out_specs=pl.BlockSpec((blk, d2), lambda i: (i, 0)),
        out_shape=jax.ShapeDtypeStruct((v, d2), jnp.float32),
    )(table_t, w2, b2)


def _gather_sc(t2, idx3, total_rows, d2):
    nchunks = idx3.shape[1]
    ngroups = nchunks // K
    rows_per_w = nchunks * CHUNK

    mesh = plsc.VectorSubcoreMesh(core_axis_name="c", subcore_axis_name="s")

    @functools.partial(
        pl.kernel,
        out_type=jax.ShapeDtypeStruct((total_rows, d2), jnp.float32),
        mesh=mesh,
        scratch_types=[
            pltpu.VMEM((nchunks, CHUNK), jnp.int32),
            pltpu.VMEM((GROUP, d2), jnp.float32),
            pltpu.SemaphoreType.DMA,
        ],
        compiler_params=pltpu.CompilerParams(use_tc_tiling_on_sc=True),
    )
    def gather_kernel(t2_hbm, idx_hbm, out_hbm, idx_v, buf, sem):
        wid = lax.axis_index("s") * NC + lax.axis_index("c")
        base = wid * rows_per_w
        pltpu.sync_copy(idx_hbm.at[wid], idx_v)

        def body(g, carry):
            copies = [
                pltpu.async_copy(
                    t2_hbm.at[idx_v.at[g * K + j]],
                    buf.at[pl.ds(j * CHUNK, CHUNK)],
                    sem,
                )
                for j in range(K)
            ]
            for c in copies:
                c.wait()
            pltpu.sync_copy(buf, out_hbm.at[pl.ds(base + g * GROUP, GROUP)])
            return carry

        lax.fori_loop(0, ngroups, body, 0)

    return gather_kernel(t2, idx3)


def _transpose_tc(g2_s, ident, acc, l0, nl, seq, d, bsz):
    blk = 8192
    jgrid = bsz // blk

    def body(*refs):
        x_ref, i_ref, o_ref = refs[-3], refs[-2], refs[-1]
        x64 = x_ref[...][:, :d]
        z = lax.dot_general(
            i_ref[...], x64,
            dimension_numbers=(((1,), (1,)), ((), ())),
            preferred_element_type=jnp.float32,
        )
        o_ref[0] = z

    in_specs = [
        pl.BlockSpec((blk, 2 * d), lambda l, j: (l * jgrid + j, 0)),
        pl.BlockSpec((d, d), lambda l, j: (0, 0)),
    ]
    args = (g2_s, ident)
    aliases = {}
    if acc is not None:
        in_specs = [pl.BlockSpec(memory_space=pl.ANY)] + in_specs
        args = (acc,) + args
        aliases = {0: 0}

    return pl.pallas_call(
        body,
        grid=(nl, jgrid),
        in_specs=in_specs,
        out_specs=pl.BlockSpec((1, d, blk), lambda l, j: (l0 + l, 0, j)),
        out_shape=jax.ShapeDtypeStruct((seq, d, bsz), jnp.float32),
        input_output_aliases=aliases,
    )(*args)


def kernel(idx, table, W, b):
    bsz, seq = idx.shape
    v, d = table.shape
    d2 = 2 * d
    m = bsz * seq
    wt = W.T
    w2 = jnp.concatenate([wt, wt], axis=1)
    b2 = jnp.concatenate([b, b]).reshape(1, d2)
    t2 = _transform_tc(table.T, w2, b2, v, d, d2)
    nsl = 5
    nl = seq // nsl
    rows_s = nl * bsz
    idx4 = idx.T.reshape(nsl, NW, rows_s // (NW * CHUNK), CHUNK).astype(jnp.int32)
    ident = jnp.eye(d, dtype=jnp.float32)
    acc = None
    for s in range(nsl):
        g2_s = _gather_sc(t2, idx4[s], rows_s, d2)
        acc = _transpose_tc(g2_s, ident, acc, s * nl, nl, seq, d, bsz)
    return jnp.transpose(acc, (2, 0, 1))


# speedup vs baseline: 2.8173x; 1.0826x over previous
import functools

import jax
import jax.numpy as jnp
from jax import lax
from jax.experimental import pallas as pl
from jax.experimental.pallas import tpu as pltpu
from jax.experimental.pallas import tpu_sc as plsc

PHI = 1.61803398875

NC, NS = 2, 16
NW = NC * NS
CHUNK = 128
K = 4
GROUP = CHUNK * K
TBLK = 4096
NTB = 124
V2 = NTB * TBLK


def _transform_tc(table_t, wt, b1, v, d):
    last_blk = (v - 1) // TBLK

    def body(xl_ref, xh_ref, w_ref, b_ref, o_ref):
        def half(x):
            acc = lax.dot_general(
                x, w_ref[...],
                dimension_numbers=(((0,), (0,)), ((), ())),
                preferred_element_type=jnp.float32,
            )
            return (acc + b_ref[...]) * PHI

        o_ref[:, :d] = half(xl_ref[...])
        o_ref[:, d:] = half(xh_ref[...])

    return pl.pallas_call(
        body,
        grid=(NTB,),
        in_specs=[
            pl.BlockSpec((d, TBLK), lambda i: (0, i)),
            pl.BlockSpec((d, TBLK), lambda i: (0, jnp.minimum(i + NTB, last_blk))),
            pl.BlockSpec((d, d), lambda i: (0, 0)),
            pl.BlockSpec((1, d), lambda i: (0, 0)),
        ],
        out_specs=pl.BlockSpec((TBLK, 2 * d), lambda i: (i, 0)),
        out_shape=jax.ShapeDtypeStruct((V2, 2 * d), jnp.float32),
    )(table_t, table_t, wt, b1)


def _gather_sc(t2, idx3, total_rows, d2):
    nchunks = idx3.shape[1]
    ngroups = nchunks // K
    rows_per_w = nchunks * CHUNK

    mesh = plsc.VectorSubcoreMesh(core_axis_name="c", subcore_axis_name="s")

    @functools.partial(
        pl.kernel,
        out_type=jax.ShapeDtypeStruct((total_rows, d2), jnp.float32),
        mesh=mesh,
        scratch_types=[
            pltpu.VMEM((nchunks, CHUNK), jnp.int32),
            pltpu.VMEM((GROUP, d2), jnp.float32),
            pltpu.SemaphoreType.DMA,
        ],
        compiler_params=pltpu.CompilerParams(use_tc_tiling_on_sc=True),
    )
    def gather_kernel(t2_hbm, idx_hbm, out_hbm, idx_v, buf, sem):
        wid = lax.axis_index("s") * NC + lax.axis_index("c")
        base = wid * rows_per_w
        pltpu.sync_copy(idx_hbm.at[wid], idx_v)

        def body(g, carry):
            copies = [
                pltpu.async_copy(
                    t2_hbm.at[idx_v.at[g * K + j]],
                    buf.at[pl.ds(j * CHUNK, CHUNK)],
                    sem,
                )
                for j in range(K)
            ]
            for c in copies:
                c.wait()
            pltpu.sync_copy(buf, out_hbm.at[pl.ds(base + g * GROUP, GROUP)])
            return carry

        lax.fori_loop(0, ngroups, body, 0)

    return gather_kernel(t2, idx3)


def _transpose_tc(g2_s, ident, m3, acc, l0, nl, seq, d, bsz):
    blk = 8192
    jgrid = bsz // blk

    def body(*refs):
        x_ref, i_ref, m_ref, o_ref = refs[-4], refs[-3], refs[-2], refs[-1]
        x = x_ref[...]

        def tr(xh):
            return lax.dot_general(
                i_ref[...], xh,
                dimension_numbers=(((1,), (1,)), ((), ())),
                preferred_element_type=jnp.float32,
            )

        zl = tr(x[:, :d])
        zr = tr(x[:, d:])
        m = m_ref[0]
        o_ref[0] = zl + (zr - zl) * m

    in_specs = [
        pl.BlockSpec((blk, 2 * d), lambda l, j: (l * jgrid + j, 0)),
        pl.BlockSpec((d, d), lambda l, j: (0, 0)),
        pl.BlockSpec((1, 1, blk), lambda l, j: (l0 + l, 0, j)),
    ]
    args = (g2_s, ident, m3)
    aliases = {}
    if acc is not None:
        in_specs = [pl.BlockSpec(memory_space=pl.ANY)] + in_specs
        args = (acc,) + args
        aliases = {0: 0}

    return pl.pallas_call(
        body,
        grid=(nl, jgrid),
        in_specs=in_specs,
        out_specs=pl.BlockSpec((1, d, blk), lambda l, j: (l0 + l, 0, j)),
        out_shape=jax.ShapeDtypeStruct((seq, d, bsz), jnp.float32),
        input_output_aliases=aliases,
    )(*args)


def kernel(idx, table, W, b):
    bsz, seq = idx.shape
    v, d = table.shape
    m = bsz * seq
    t2 = _transform_tc(table.T, W.T, b.reshape(1, d), v, d)
    idx_t = idx.T
    hi = idx_t >= V2
    idx_g = (idx_t - V2 * hi.astype(idx_t.dtype)).astype(jnp.int32)
    m3 = hi.astype(jnp.float32).reshape(seq, 1, bsz)
    nsl = 5
    nl = seq // nsl
    rows_s = nl * bsz
    idx4 = idx_g.reshape(nsl, NW, rows_s // (NW * CHUNK), CHUNK)
    ident = jnp.eye(d, dtype=jnp.float32)
    acc = None
    for s in range(nsl):
        g2_s = _gather_sc(t2, idx4[s], rows_s, 2 * d)
        acc = _transpose_tc(g2_s, ident, m3, acc, s * nl, nl, seq, d, bsz)
    return jnp.transpose(acc, (2, 0, 1))


# speedup vs baseline: 2.9350x; 1.0418x over previous
import functools

import jax
import jax.numpy as jnp
from jax import lax
from jax.experimental import pallas as pl
from jax.experimental.pallas import tpu as pltpu
from jax.experimental.pallas import tpu_sc as plsc

PHI = 1.61803398875

NC, NS = 2, 16
NW = NC * NS
CHUNK = 128
K = 5
GROUP = CHUNK * K
TBLK = 8192
NTB = 62
V2 = NTB * TBLK


def _transform_tc(table_t, wt, b1, v, d):
    last_blk = (v - 1) // TBLK

    def body(xl_ref, xh_ref, w_ref, b_ref, o_ref):
        def half(x):
            acc = lax.dot_general(
                x, w_ref[...],
                dimension_numbers=(((0,), (0,)), ((), ())),
                preferred_element_type=jnp.float32,
            )
            return (acc + b_ref[...]) * PHI

        o_ref[:, :d] = half(xl_ref[...])
        o_ref[:, d:] = half(xh_ref[...])

    return pl.pallas_call(
        body,
        grid=(NTB,),
        in_specs=[
            pl.BlockSpec((d, TBLK), lambda i: (0, i)),
            pl.BlockSpec((d, TBLK), lambda i: (0, jnp.minimum(i + NTB, last_blk))),
            pl.BlockSpec((d, d), lambda i: (0, 0)),
            pl.BlockSpec((1, d), lambda i: (0, 0)),
        ],
        out_specs=pl.BlockSpec((TBLK, 2 * d), lambda i: (i, 0)),
        out_shape=jax.ShapeDtypeStruct((V2, 2 * d), jnp.float32),
    )(table_t, table_t, wt, b1)


def _gather_sc(t2, idx3, total_rows, d2):
    nchunks = idx3.shape[1]
    ngroups = nchunks // K
    rows_per_w = nchunks * CHUNK

    mesh = plsc.VectorSubcoreMesh(core_axis_name="c", subcore_axis_name="s")

    @functools.partial(
        pl.kernel,
        out_type=jax.ShapeDtypeStruct((total_rows, d2), jnp.float32),
        mesh=mesh,
        scratch_types=[
            pltpu.VMEM((nchunks, CHUNK), jnp.int32),
            pltpu.VMEM((GROUP, d2), jnp.float32),
            pltpu.SemaphoreType.DMA,
        ],
        compiler_params=pltpu.CompilerParams(use_tc_tiling_on_sc=True),
    )
    def gather_kernel(t2_hbm, idx_hbm, out_hbm, idx_v, buf, sem):
        wid = lax.axis_index("s") * NC + lax.axis_index("c")
        base = wid * rows_per_w
        pltpu.sync_copy(idx_hbm.at[wid], idx_v)

        def body(g, carry):
            copies = [
                pltpu.async_copy(
                    t2_hbm.at[idx_v.at[g * K + j]],
                    buf.at[pl.ds(j * CHUNK, CHUNK)],
                    sem,
                )
                for j in range(K)
            ]
            for c in copies:
                c.wait()
            pltpu.sync_copy(buf, out_hbm.at[pl.ds(base + g * GROUP, GROUP)])
            return carry

        lax.fori_loop(0, ngroups, body, 0)

    return gather_kernel(t2, idx3)


def _transpose_tc(g2_s, ident, m3, acc, l0, nl, seq, d, bsz):
    blk = 8192
    jgrid = bsz // blk

    def body(*refs):
        x_ref, i_ref, m_ref, o_ref = refs[-4], refs[-3], refs[-2], refs[-1]
        x = x_ref[...]

        def tr(xh):
            return lax.dot_general(
                i_ref[...], xh,
                dimension_numbers=(((1,), (1,)), ((), ())),
                preferred_element_type=jnp.float32,
            )

        zl = tr(x[:, :d])
        zr = tr(x[:, d:])
        m = m_ref[0]
        o_ref[0] = zl + (zr - zl) * m

    in_specs = [
        pl.BlockSpec((blk, 2 * d), lambda l, j: (l * jgrid + j, 0)),
        pl.BlockSpec((d, d), lambda l, j: (0, 0)),
        pl.BlockSpec((1, 1, blk), lambda l, j: (l0 + l, 0, j)),
    ]
    args = (g2_s, ident, m3)
    aliases = {}
    if acc is not None:
        in_specs = [pl.BlockSpec(memory_space=pl.ANY)] + in_specs
        args = (acc,) + args
        aliases = {0: 0}

    return pl.pallas_call(
        body,
        grid=(nl, jgrid),
        in_specs=in_specs,
        out_specs=pl.BlockSpec((1, d, blk), lambda l, j: (l0 + l, 0, j)),
        out_shape=jax.ShapeDtypeStruct((seq, d, bsz), jnp.float32),
        input_output_aliases=aliases,
    )(*args)


def kernel(idx, table, W, b):
    bsz, seq = idx.shape
    v, d = table.shape
    m = bsz * seq
    t2 = _transform_tc(table.T, W.T, b.reshape(1, d), v, d)
    idx_t = idx.T
    hi = idx_t >= V2
    idx_g = (idx_t - V2 * hi.astype(idx_t.dtype)).astype(jnp.int32)
    m3 = hi.astype(jnp.float32).reshape(seq, 1, bsz)
    nsl = 5
    nl = seq // nsl
    rows_s = nl * bsz
    idx4 = idx_g.reshape(nsl, NW, rows_s // (NW * CHUNK), CHUNK)
    ident = jnp.eye(d, dtype=jnp.float32)
    acc = None
    for s in range(nsl):
        g2_s = _gather_sc(t2, idx4[s], rows_s, 2 * d)
        acc = _transpose_tc(g2_s, ident, m3, acc, s * nl, nl, seq, d, bsz)
    return jnp.transpose(acc, (2, 0, 1))


# speedup vs baseline: 2.9743x; 1.0134x over previous
import functools

import jax
import jax.numpy as jnp
from jax import lax
from jax.experimental import pallas as pl
from jax.experimental.pallas import tpu as pltpu
from jax.experimental.pallas import tpu_sc as plsc

PHI = 1.61803398875

NC, NS = 2, 16
NW = NC * NS
CHUNK = 128
K = 5
GROUP = CHUNK * K
TBLK = 8192
NTB = 62
V2 = NTB * TBLK


def _transform_tc(table_t, wt, b1, v, d):
    last_blk = (v - 1) // TBLK

    def body(xl_ref, xh_ref, w_ref, b_ref, o_ref):
        def half(x):
            acc = lax.dot_general(
                x, w_ref[...],
                dimension_numbers=(((0,), (0,)), ((), ())),
                preferred_element_type=jnp.float32,
            )
            return (acc + b_ref[...]) * PHI

        o_ref[:, :d] = half(xl_ref[...])
        o_ref[:, d:] = half(xh_ref[...])

    return pl.pallas_call(
        body,
        grid=(NTB,),
        in_specs=[
            pl.BlockSpec((d, TBLK), lambda i: (0, i)),
            pl.BlockSpec((d, TBLK), lambda i: (0, jnp.minimum(i + NTB, last_blk))),
            pl.BlockSpec((d, d), lambda i: (0, 0)),
            pl.BlockSpec((1, d), lambda i: (0, 0)),
        ],
        out_specs=pl.BlockSpec((TBLK, 2 * d), lambda i: (i, 0)),
        out_shape=jax.ShapeDtypeStruct((V2, 2 * d), jnp.float32),
    )(table_t, table_t, wt, b1)


def _gather_sc(t2, idx3, total_rows, d2):
    nchunks = idx3.shape[1]
    ngroups = nchunks // K
    rows_per_w = nchunks * CHUNK

    mesh = plsc.VectorSubcoreMesh(core_axis_name="c", subcore_axis_name="s")

    @functools.partial(
        pl.kernel,
        out_type=jax.ShapeDtypeStruct((total_rows, d2), jnp.float32),
        mesh=mesh,
        scratch_types=[
            pltpu.VMEM((nchunks, CHUNK), jnp.int32),
            pltpu.VMEM((GROUP, d2), jnp.float32),
            pltpu.SemaphoreType.DMA,
            pltpu.SemaphoreType.DMA,
        ],
        compiler_params=pltpu.CompilerParams(use_tc_tiling_on_sc=True),
    )
    def gather_kernel(t2_hbm, idx_hbm, out_hbm, idx_v, buf, sem, semf):
        wid = lax.axis_index("s") * NC + lax.axis_index("c")
        base = wid * rows_per_w
        h1 = 2 * CHUNK
        h2 = GROUP - h1
        pltpu.sync_copy(idx_hbm.at[wid], idx_v)

        def drain_flushes():
            pltpu.make_async_copy(
                buf.at[pl.ds(0, h1)], out_hbm.at[pl.ds(base, h1)], semf
            ).wait()
            pltpu.make_async_copy(
                buf.at[pl.ds(h1, h2)], out_hbm.at[pl.ds(base, h2)], semf
            ).wait()

        def body(g, carry):
            @pl.when(g > 0)
            def _():
                drain_flushes()

            off = base + g * GROUP
            copies = [
                pltpu.async_copy(
                    t2_hbm.at[idx_v.at[g * K + j]],
                    buf.at[pl.ds(j * CHUNK, CHUNK)],
                    sem,
                )
                for j in range(K)
            ]
            copies[0].wait()
            copies[1].wait()
            pltpu.async_copy(
                buf.at[pl.ds(0, h1)], out_hbm.at[pl.ds(off, h1)], semf
            )
            for c in copies[2:]:
                c.wait()
            pltpu.async_copy(
                buf.at[pl.ds(h1, h2)], out_hbm.at[pl.ds(off + h1, h2)], semf
            )
            return carry

        lax.fori_loop(0, ngroups, body, 0)
        drain_flushes()

    return gather_kernel(t2, idx3)


def _transpose_tc(g2_s, ident, m3, acc, l0, nl, seq, d, bsz):
    blk = 8192
    jgrid = bsz // blk

    def body(*refs):
        x_ref, i_ref, m_ref, o_ref = refs[-4], refs[-3], refs[-2], refs[-1]
        x = x_ref[...]

        def tr(xh):
            return lax.dot_general(
                i_ref[...], xh,
                dimension_numbers=(((1,), (1,)), ((), ())),
                preferred_element_type=jnp.float32,
            )

        zl = tr(x[:, :d])
        zr = tr(x[:, d:])
        m = m_ref[0]
        o_ref[0] = zl + (zr - zl) * m

    in_specs = [
        pl.BlockSpec((blk, 2 * d), lambda l, j: (l * jgrid + j, 0)),
        pl.BlockSpec((d, d), lambda l, j: (0, 0)),
        pl.BlockSpec((1, 1, blk), lambda l, j: (l0 + l, 0, j)),
    ]
    args = (g2_s, ident, m3)
    aliases = {}
    if acc is not None:
        in_specs = [pl.BlockSpec(memory_space=pl.ANY)] + in_specs
        args = (acc,) + args
        aliases = {0: 0}

    return pl.pallas_call(
        body,
        grid=(nl, jgrid),
        in_specs=in_specs,
        out_specs=pl.BlockSpec((1, d, blk), lambda l, j: (l0 + l, 0, j)),
        out_shape=jax.ShapeDtypeStruct((seq, d, bsz), jnp.float32),
        input_output_aliases=aliases,
    )(*args)


def kernel(idx, table, W, b):
    bsz, seq = idx.shape
    v, d = table.shape
    m = bsz * seq
    t2 = _transform_tc(table.T, W.T, b.reshape(1, d), v, d)
    idx_t = idx.T
    hi = idx_t >= V2
    idx_g = (idx_t - V2 * hi.astype(idx_t.dtype)).astype(jnp.int32)
    m3 = hi.astype(jnp.float32).reshape(seq, 1, bsz)
    nsl = 5
    nl = seq // nsl
    rows_s = nl * bsz
    idx4 = idx_g.reshape(nsl, NW, rows_s // (NW * CHUNK), CHUNK)
    ident = jnp.eye(d, dtype=jnp.float32)
    acc = None
    for s in range(nsl):
        g2_s = _gather_sc(t2, idx4[s], rows_s, 2 * d)
        acc = _transpose_tc(g2_s, ident, m3, acc, s * nl, nl, seq, d, bsz)
    return jnp.transpose(acc, (2, 0, 1))
